# baseline (speedup 1.0000x reference)
import functools

import jax
import jax.numpy as jnp
from jax import lax
from jax.experimental import pallas as pl
from jax.experimental.pallas import tpu as pltpu
from jax.experimental.pallas import tpu_sc as plsc

VOCAB = 1000000
EMB = 64
HID = 100
B = 1024
T = 200
GP = 128
NG = 4 * GP

NC = 4
TCK = T // NC

NW = 32
TOT = B * TCK
PER_W = TOT // NW
CHUNK = 80
NCH = PER_W // CHUNK


def _gather_sc(emb, idx3):
    mesh = plsc.VectorSubcoreMesh(core_axis_name="c", subcore_axis_name="s")

    @functools.partial(
        pl.kernel,
        mesh=mesh,
        out_type=jax.ShapeDtypeStruct((TOT, EMB), jnp.float32),
        scratch_types=[
            pltpu.VMEM((NCH, CHUNK), jnp.int32),
            pltpu.VMEM((CHUNK, EMB), jnp.float32),
            pltpu.VMEM((CHUNK, EMB), jnp.float32),
            pltpu.SemaphoreType.DMA,
            pltpu.SemaphoreType.DMA,
        ],
        compiler_params=pltpu.CompilerParams(use_tc_tiling_on_sc=False),
    )
    def k(emb_hbm, idx_hbm, out_hbm, idx_v, rows_a, rows_b, sem_a, sem_b):
        wid = lax.axis_index("s") * 2 + lax.axis_index("c")
        pltpu.sync_copy(idx_hbm.at[wid], idx_v)
        base = wid * PER_W

        pltpu.async_copy(emb_hbm.at[idx_v.at[0]], rows_a, sem_a)

        def body(kk, carry):
            j = 2 * kk
            pltpu.async_copy(emb_hbm.at[idx_v.at[j + 1]], rows_b, sem_b)
            pltpu.make_async_copy(emb_hbm.at[idx_v.at[j]], rows_a, sem_a).wait()
            pltpu.sync_copy(rows_a, out_hbm.at[pl.ds(base + j * CHUNK, CHUNK)])

            @pl.when(j + 2 < NCH)
            def _():
                pltpu.async_copy(emb_hbm.at[idx_v.at[j + 2]], rows_a, sem_a)

            pltpu.make_async_copy(
                emb_hbm.at[idx_v.at[j + 1]], rows_b, sem_b).wait()
            pltpu.sync_copy(
                rows_b, out_hbm.at[pl.ds(base + (j + 1) * CHUNK, CHUNK)])
            return carry

        lax.fori_loop(0, NCH // 2, body, 0)

    return k(emb, idx3)


TS = 2
NT = TCK // TS


def _make_lstm_body(last):
    def body(e_ref, wih_ref, whh_ref, b_ref, fcw_ref, fcb_ref,
             hin_ref, cin_ref, *out_and_scratch):
        if last:
            out_ref, h_ref, c_ref = out_and_scratch
        else:
            hout_ref, cout_ref, h_ref, c_ref = out_and_scratch
        tb = pl.program_id(0)

        @pl.when(tb == 0)
        def _init():
            h_ref[...] = hin_ref[...]
            c_ref[...] = cin_ref[...]

        h = h_ref[...]
        c = c_ref[...]
        for k in range(TS):
            e_t = e_ref[:, k * EMB:(k + 1) * EMB].astype(jnp.bfloat16)
            gates = (jnp.dot(e_t, wih_ref[...],
                             preferred_element_type=jnp.float32)
                     + jnp.dot(h.astype(jnp.bfloat16), whh_ref[...],
                               preferred_element_type=jnp.float32)
                     + b_ref[...])
            i = jax.nn.sigmoid(gates[:, 0:GP])
            f = jax.nn.sigmoid(gates[:, GP:2 * GP])
            g = jnp.tanh(gates[:, 2 * GP:3 * GP])
            o = jax.nn.sigmoid(gates[:, 3 * GP:4 * GP])
            c = f * c + i * g
            h = o * jnp.tanh(c)
        h_ref[...] = h
        c_ref[...] = c

        @pl.when(tb == NT - 1)
        def _fin():
            if last:
                out_ref[...] = jax.nn.sigmoid(
                    jnp.sum(h * fcw_ref[...], axis=1, keepdims=True)
                    + fcb_ref[...])
            else:
                hout_ref[...] = h
                cout_ref[...] = c

    return body


def _lstm_tc(e_bte, wih_p, whh_p, b_p, fcw_p, fcb_p, h_in, c_in, last):
    if last:
        out_shape = jax.ShapeDtypeStruct((B, 1), jnp.float32)
        out_specs = pl.BlockSpec((B, 1), lambda t: (0, 0))
    else:
        out_shape = (jax.ShapeDtypeStruct((B, GP), jnp.float32),
                     jax.ShapeDtypeStruct((B, GP), jnp.float32))
        out_specs = (pl.BlockSpec((B, GP), lambda t: (0, 0)),
                     pl.BlockSpec((B, GP), lambda t: (0, 0)))
    return pl.pallas_call(
        _make_lstm_body(last),
        grid=(NT,),
        in_specs=[
            pl.BlockSpec((B, TS * EMB), lambda t: (0, t)),
            pl.BlockSpec((EMB, NG), lambda t: (0, 0)),
            pl.BlockSpec((GP, NG), lambda t: (0, 0)),
            pl.BlockSpec((1, NG), lambda t: (0, 0)),
            pl.BlockSpec((1, GP), lambda t: (0, 0)),
            pl.BlockSpec((1, 1), lambda t: (0, 0)),
            pl.BlockSpec((B, GP), lambda t: (0, 0)),
            pl.BlockSpec((B, GP), lambda t: (0, 0)),
        ],
        out_specs=out_specs,
        out_shape=out_shape,
        scratch_shapes=[
            pltpu.VMEM((B, GP), jnp.float32),
            pltpu.VMEM((B, GP), jnp.float32),
        ],
    )(e_bte, wih_p, whh_p, b_p, fcw_p, fcb_p, h_in, c_in)


def kernel(x, emb, W_ih, W_hh, b_ih, b_hh, fc_w, fc_b):
    xi = x.astype(jnp.int32)

    w_ih4 = W_ih.reshape(4, HID, EMB)
    wih_p = jnp.zeros((4, GP, EMB), jnp.float32).at[:, :HID, :].set(w_ih4)
    wih_p = wih_p.reshape(NG, EMB).T.astype(jnp.bfloat16)
    w_hh4 = W_hh.reshape(4, HID, HID)
    whh_p = jnp.zeros((4, GP, GP), jnp.float32).at[:, :HID, :HID].set(w_hh4)
    whh_p = whh_p.reshape(NG, GP).T.astype(jnp.bfloat16)
    b4 = (b_ih + b_hh).reshape(4, HID)
    b_p = jnp.zeros((4, GP), jnp.float32).at[:, :HID].set(b4).reshape(1, NG)
    fcw_p = jnp.zeros((1, GP), jnp.float32).at[:, :HID].set(fc_w)
    fcb_p = fc_b.reshape(1, 1)

    es = []
    for kc in range(NC):
        idx3 = xi[:, kc * TCK:(kc + 1) * TCK].reshape(NW, NCH, CHUNK)
        e_k = _gather_sc(emb, idx3).reshape(B, TCK * EMB)
        es.append(lax.optimization_barrier(e_k))

    h = jnp.zeros((B, GP), jnp.float32)
    c = jnp.zeros((B, GP), jnp.float32)
    for kc in range(NC - 1):
        h, c = _lstm_tc(es[kc], wih_p, whh_p, b_p, fcw_p, fcb_p, h, c, False)
    out = _lstm_tc(es[NC - 1], wih_p, whh_p, b_p, fcw_p, fcb_p, h, c, True)
    return out.reshape(B)


# speedup vs baseline: 1.0322x; 1.0322x over previous
import functools

import jax
import jax.numpy as jnp
from jax import lax
from jax.experimental import pallas as pl
from jax.experimental.pallas import tpu as pltpu
from jax.experimental.pallas import tpu_sc as plsc

VOCAB = 1000000
EMB = 64
HID = 100
B = 1024
T = 200
GP = 128
NG = 4 * GP

NW = 32
TOT = B * T
PER_W = TOT // NW
CHUNK = 128
NCH = PER_W // CHUNK


def _gather_sc(emb, idx3):
    mesh = plsc.VectorSubcoreMesh(core_axis_name="c", subcore_axis_name="s")

    @functools.partial(
        pl.kernel,
        mesh=mesh,
        out_type=jax.ShapeDtypeStruct((TOT, EMB), jnp.float32),
        scratch_types=[
            pltpu.VMEM((NCH, CHUNK), jnp.int32),
            pltpu.VMEM((CHUNK, EMB), jnp.float32),
            pltpu.VMEM((CHUNK, EMB), jnp.float32),
            pltpu.SemaphoreType.DMA,
            pltpu.SemaphoreType.DMA,
        ],
        compiler_params=pltpu.CompilerParams(use_tc_tiling_on_sc=False),
    )
    def k(emb_hbm, idx_hbm, out_hbm, idx_v, rows_a, rows_b, sem_a, sem_b):
        wid = lax.axis_index("s") * 2 + lax.axis_index("c")
        pltpu.sync_copy(idx_hbm.at[wid], idx_v)
        base = wid * PER_W

        pltpu.async_copy(emb_hbm.at[idx_v.at[0]], rows_a, sem_a)

        def body(kk, carry):
            j = 2 * kk
            pltpu.async_copy(emb_hbm.at[idx_v.at[j + 1]], rows_b, sem_b)
            pltpu.make_async_copy(emb_hbm.at[idx_v.at[j]], rows_a, sem_a).wait()
            pltpu.sync_copy(rows_a, out_hbm.at[pl.ds(base + j * CHUNK, CHUNK)])

            @pl.when(j + 2 < NCH)
            def _():
                pltpu.async_copy(emb_hbm.at[idx_v.at[j + 2]], rows_a, sem_a)

            pltpu.make_async_copy(
                emb_hbm.at[idx_v.at[j + 1]], rows_b, sem_b).wait()
            pltpu.sync_copy(
                rows_b, out_hbm.at[pl.ds(base + (j + 1) * CHUNK, CHUNK)])
            return carry

        lax.fori_loop(0, NCH // 2, body, 0)

    return k(emb, idx3)


TS = 4
NT = T // TS


def _lstm_body(e_ref, wih_ref, whh_ref, b_ref, fcw_ref, fcb_ref,
               out_ref, h_ref, c_ref):
    tb = pl.program_id(0)

    @pl.when(tb == 0)
    def _init():
        h_ref[...] = jnp.zeros_like(h_ref)
        c_ref[...] = jnp.zeros_like(c_ref)

    h = h_ref[...]
    c = c_ref[...]
    for k in range(TS):
        e_t = e_ref[:, k * EMB:(k + 1) * EMB].astype(jnp.bfloat16)
        gates = (jnp.dot(e_t, wih_ref[...], preferred_element_type=jnp.float32)
                 + jnp.dot(h.astype(jnp.bfloat16), whh_ref[...],
                           preferred_element_type=jnp.float32)
                 + b_ref[...])
        i = jax.nn.sigmoid(gates[:, 0:GP])
        f = jax.nn.sigmoid(gates[:, GP:2 * GP])
        g = jnp.tanh(gates[:, 2 * GP:3 * GP])
        o = jax.nn.sigmoid(gates[:, 3 * GP:4 * GP])
        c = f * c + i * g
        h = o * jnp.tanh(c)
    h_ref[...] = h
    c_ref[...] = c

    @pl.when(tb == NT - 1)
    def _head():
        out_ref[...] = jax.nn.sigmoid(
            jnp.sum(h * fcw_ref[...], axis=1, keepdims=True) + fcb_ref[...])


def _lstm_tc(e_bte, wih_p, whh_p, b_p, fcw_p, fcb_p):
    return pl.pallas_call(
        _lstm_body,
        grid=(NT,),
        in_specs=[
            pl.BlockSpec((B, TS * EMB), lambda t: (0, t)),
            pl.BlockSpec((EMB, NG), lambda t: (0, 0)),
            pl.BlockSpec((GP, NG), lambda t: (0, 0)),
            pl.BlockSpec((1, NG), lambda t: (0, 0)),
            pl.BlockSpec((1, GP), lambda t: (0, 0)),
            pl.BlockSpec((1, 1), lambda t: (0, 0)),
        ],
        out_specs=pl.BlockSpec((B, 1), lambda t: (0, 0)),
        out_shape=jax.ShapeDtypeStruct((B, 1), jnp.float32),
        scratch_shapes=[
            pltpu.VMEM((B, GP), jnp.float32),
            pltpu.VMEM((B, GP), jnp.float32),
        ],
    )(e_bte, wih_p, whh_p, b_p, fcw_p, fcb_p)


def kernel(x, emb, W_ih, W_hh, b_ih, b_hh, fc_w, fc_b):
    idx3 = x.astype(jnp.int32).reshape(NW, NCH, CHUNK)
    e = _gather_sc(emb, idx3).reshape(B, T * EMB)

    w_ih4 = W_ih.reshape(4, HID, EMB)
    wih_p = jnp.zeros((4, GP, EMB), jnp.float32).at[:, :HID, :].set(w_ih4)
    wih_p = wih_p.reshape(NG, EMB).T.astype(jnp.bfloat16)
    w_hh4 = W_hh.reshape(4, HID, HID)
    whh_p = jnp.zeros((4, GP, GP), jnp.float32).at[:, :HID, :HID].set(w_hh4)
    whh_p = whh_p.reshape(NG, GP).T.astype(jnp.bfloat16)
    b4 = (b_ih + b_hh).reshape(4, HID)
    b_p = jnp.zeros((4, GP), jnp.float32).at[:, :HID].set(b4).reshape(1, NG)
    fcw_p = jnp.zeros((1, GP), jnp.float32).at[:, :HID].set(fc_w)
    fcb_p = fc_b.reshape(1, 1)

    out = _lstm_tc(e, wih_p, whh_p, b_p, fcw_p, fcb_p)
    return out.reshape(B)
